# baseline (speedup 1.0000x reference)
import numpy as np
import jax
import jax.numpy as jnp
from jax import lax
from jax.experimental import pallas as pl
from jax.experimental.pallas import tpu as pltpu
from jax.experimental.pallas import tpu_sc as plsc

_MIN_LOC = np.array([0.0, 0.0], dtype=np.float32)
_MAX_LOC = np.array([1.0, 1.0], dtype=np.float32)
_BINS = np.array([1023.0, 1023.0], dtype=np.float32)
_STRIDE = np.array([4.0, 4.0], dtype=np.float32)
_WINDOW = np.array([1.0, 1.0], dtype=np.float32)
_BATCH, _NLOC = 64, 200


def _build_base():
    window_side = (_WINDOW / 2.0).astype(np.int32).astype(np.float32)
    loc_delta = (_MAX_LOC - _MIN_LOC) / _BINS
    bins_window = _BINS - 2.0 * window_side
    min_window = _MIN_LOC + loc_delta * window_side
    max_window = _MIN_LOC + loc_delta * bins_window
    bins_stride = ((bins_window + 1.0) / _STRIDE).astype(np.int32)
    delta2 = (max_window - min_window) / bins_stride.astype(np.float32)
    xs = np.arange(min_window[0], max_window[0], delta2[0], dtype=np.float32)
    ys = np.arange(min_window[1], max_window[1], delta2[1], dtype=np.float32)
    X, Y = np.meshgrid(xs, ys)
    loc_base = np.stack([X, Y], axis=-1).astype(np.float32)
    corr_base = np.full(loc_base.shape, 0.634, dtype=np.float32)
    base4 = np.concatenate([corr_base, loc_base], axis=-1)
    return base4, loc_delta


_BASE4, _LOC_DELTA = _build_base()
_G = _BASE4.shape[0]
_DL0 = np.float32(_LOC_DELTA[0])
_DL1 = np.float32(_LOC_DELTA[1])
_K = np.float32(255.75)


def _ref_idx(x, dl):
    return ((x / dl) / np.float32(4.0)).astype(np.int32)


def _build_boundaries(dl):
    B = np.zeros(257, dtype=np.float32)
    for k in range(1, 257):
        x = np.float32(np.float64(k) * 4.0 * np.float64(dl))
        while x > 0:
            nx = np.nextafter(x, np.float32(0.0), dtype=np.float32)
            if _ref_idx(nx, dl) >= k:
                x = nx
            else:
                break
        while _ref_idx(x, dl) < k:
            x = np.nextafter(x, np.float32(2.0), dtype=np.float32)
        B[k] = x
    B[0] = 0.0
    return B


def _build_btab():
    assert _DL0 == _DL1 and float(_STRIDE[0]) == float(_STRIDE[1]) == 4.0
    B = _build_boundaries(_DL0)
    tab = np.full((8, 128), 2.0, dtype=np.float32)
    flat = tab.reshape(-1)
    flat[0:257] = B
    flat[512:768] = B[1:]
    return tab


_BTAB = _build_btab()

_NP = 224
_NSLICE = 4
_SLICE_ROWS = _G // _NSLICE
_ROWS = _G
_SAMPLES_PER_W = 2


def _sc_body(locs_hbm, base_hbm, btab_hbm, out_hbm, locs_v, btab_v, addr16,
             vals16, mapbuf):
    wid = lax.axis_index("s") * 2 + lax.axis_index("c")
    iota = lax.iota(jnp.int32, 16)
    one_f = jnp.ones((16,), jnp.float32)
    pltpu.sync_copy(btab_hbm, btab_v)

    def bin_idx(x):
        q = (x * _K).astype(jnp.int32)
        lo = plsc.load_gather(btab_v, [q >> 7, q & 127])
        qh = q + 512
        hi = plsc.load_gather(btab_v, [qh >> 7, qh & 127])
        return q + (x >= hi).astype(jnp.int32) - (x < lo).astype(jnp.int32)

    for t in range(_SAMPLES_PER_W):
        b = wid * _SAMPLES_PER_W + t
        pltpu.sync_copy(locs_hbm.at[b], locs_v)

        for k in range(_NP // 16):
            f = k * 16
            l0 = locs_v[f // 128, pl.ds(f % 128, 16)]
            l1 = locs_v[4 + f // 128, pl.ds(f % 128, 16)]
            px = bin_idx(l0)
            py = bin_idx(l1)
            rm = jnp.where(px < 1, px + (_G - 1), px - 1)
            cm = jnp.where(py < 1, py + (_G - 1), py - 1)
            cells = (rm * 1024 + (cm >> 7) * 512 + (cm & 127),
                     rm * 1024 + (py >> 7) * 512 + (py & 127),
                     px * 1024 + (cm >> 7) * 512 + (cm & 127),
                     px * 1024 + (py >> 7) * 512 + (py & 127))
            vals = (one_f, one_f, l0, l1)
            pos = (k * 16 + iota) * 16
            for c4 in range(4):
                for ch in range(4):
                    plsc.store_scatter(addr16, [pos + (c4 * 4 + ch)],
                                       cells[c4] + ch * 128)
                    plsc.store_scatter(vals16, [pos + (c4 * 4 + ch)], vals[ch])

        for q in range(_NSLICE):
            pltpu.sync_copy(base_hbm.at[pl.ds(q * _SLICE_ROWS, _SLICE_ROWS)],
                            mapbuf)

            def patch(i, carry, q=q):
                a = addr16[pl.ds(i * 16, 16)]
                v = vals16[pl.ds(i * 16, 16)]
                mask = (a >> 16) == q
                r = (a >> 10) - q * _SLICE_ROWS
                plsc.store_scatter(mapbuf, [r, (a >> 7) & 7, a & 127], v,
                                   mask=mask)
                return carry

            lax.fori_loop(0, _NP, patch, 0)
            pltpu.sync_copy(
                mapbuf,
                out_hbm.at[pl.ds(b * _ROWS + q * _SLICE_ROWS, _SLICE_ROWS)])


_INTERPRET = False


def _sc_call(locs, base, btab):
    mesh = plsc.VectorSubcoreMesh(core_axis_name="c", subcore_axis_name="s")
    return pl.kernel(
        _sc_body,
        out_type=jax.ShapeDtypeStruct((_BATCH * _ROWS, 8, 128), jnp.float32),
        mesh=mesh,
        scratch_types=[
            pltpu.VMEM((8, 128), jnp.float32),
            pltpu.VMEM((8, 128), jnp.float32),
            pltpu.VMEM((_NP * 16,), jnp.int32),
            pltpu.VMEM((_NP * 16,), jnp.float32),
            pltpu.VMEM((_SLICE_ROWS, 8, 128), jnp.float32),
        ],
        compiler_params=pltpu.CompilerParams(needs_layout_passes=False),
        interpret=_INTERPRET,
    )(locs, base, btab)


def kernel(batch):
    l0 = jnp.pad(batch[:, :, 0], ((0, 0), (0, 56)), mode="edge")
    l1 = jnp.pad(batch[:, :, 1], ((0, 0), (0, 56)), mode="edge")
    z = jnp.zeros_like(l0)
    locs = jnp.concatenate([l0, z, l1, z], axis=1).reshape(_BATCH, 8, 128)
    base_phys = np.transpose(
        _BASE4.reshape(_G, 2, 128, 4), (0, 1, 3, 2)).reshape(_G, 8, 128)
    out = _sc_call(locs, jnp.asarray(base_phys), jnp.asarray(_BTAB))
    out = out.reshape(_BATCH, _G, 2, 4, 128).transpose(0, 1, 2, 4, 3)
    return out.reshape(_BATCH, _G, _G, 4)


# speedup vs baseline: 1.0842x; 1.0842x over previous
import numpy as np
import jax
import jax.numpy as jnp
from jax import lax
from jax.experimental import pallas as pl
from jax.experimental.pallas import tpu as pltpu
from jax.experimental.pallas import tpu_sc as plsc

_MIN_LOC = np.array([0.0, 0.0], dtype=np.float32)
_MAX_LOC = np.array([1.0, 1.0], dtype=np.float32)
_BINS = np.array([1023.0, 1023.0], dtype=np.float32)
_STRIDE = np.array([4.0, 4.0], dtype=np.float32)
_WINDOW = np.array([1.0, 1.0], dtype=np.float32)
_BATCH, _NLOC = 64, 200


def _build_base():
    window_side = (_WINDOW / 2.0).astype(np.int32).astype(np.float32)
    loc_delta = (_MAX_LOC - _MIN_LOC) / _BINS
    bins_window = _BINS - 2.0 * window_side
    min_window = _MIN_LOC + loc_delta * window_side
    max_window = _MIN_LOC + loc_delta * bins_window
    bins_stride = ((bins_window + 1.0) / _STRIDE).astype(np.int32)
    delta2 = (max_window - min_window) / bins_stride.astype(np.float32)
    xs = np.arange(min_window[0], max_window[0], delta2[0], dtype=np.float32)
    ys = np.arange(min_window[1], max_window[1], delta2[1], dtype=np.float32)
    X, Y = np.meshgrid(xs, ys)
    loc_base = np.stack([X, Y], axis=-1).astype(np.float32)
    corr_base = np.full(loc_base.shape, 0.634, dtype=np.float32)
    base4 = np.concatenate([corr_base, loc_base], axis=-1)
    return base4, loc_delta


_BASE4, _LOC_DELTA = _build_base()
_G = _BASE4.shape[0]
_DL0 = np.float32(_LOC_DELTA[0])
_DL1 = np.float32(_LOC_DELTA[1])
_K = np.float32(255.75)


def _ref_idx(x, dl):
    return ((x / dl) / np.float32(4.0)).astype(np.int32)


def _build_boundaries(dl):
    B = np.zeros(257, dtype=np.float32)
    for k in range(1, 257):
        x = np.float32(np.float64(k) * 4.0 * np.float64(dl))
        while x > 0:
            nx = np.nextafter(x, np.float32(0.0), dtype=np.float32)
            if _ref_idx(nx, dl) >= k:
                x = nx
            else:
                break
        while _ref_idx(x, dl) < k:
            x = np.nextafter(x, np.float32(2.0), dtype=np.float32)
        B[k] = x
    B[0] = 0.0
    return B


def _build_btab():
    assert _DL0 == _DL1 and float(_STRIDE[0]) == float(_STRIDE[1]) == 4.0
    B = _build_boundaries(_DL0)
    tab = np.full((8, 128), 2.0, dtype=np.float32)
    flat = tab.reshape(-1)
    flat[0:257] = B
    flat[512:768] = B[1:]
    return tab


_BTAB = _build_btab()

_NP = 224
_NSLICE = 8
_SLICE_ROWS = _G // _NSLICE
_ROWS = _G
_SAMPLES_PER_W = 2


def _sc_body(locs_hbm, base_hbm, btab_hbm, out_hbm, locs_v, btab_v, addr16,
             vals16, buf0, buf1, buf2, fs0, fs1, fs2, os0, os1, os2):
    bufs = (buf0, buf1, buf2)
    fsems = (fs0, fs1, fs2)
    osems = (os0, os1, os2)
    wid = lax.axis_index("s") * 2 + lax.axis_index("c")
    iota = lax.iota(jnp.int32, 16)
    one_f = jnp.ones((16,), jnp.float32)
    pltpu.sync_copy(btab_hbm, btab_v)

    fills = [None, None, None]
    outs = [None, None, None]

    def bin_idx(x):
        q = (x * _K).astype(jnp.int32)
        lo = plsc.load_gather(btab_v, [q >> 7, q & 127])
        qh = q + 512
        hi = plsc.load_gather(btab_v, [qh >> 7, qh & 127])
        return q + (x >= hi).astype(jnp.int32) - (x < lo).astype(jnp.int32)

    for t in range(_SAMPLES_PER_W):
        b = wid * _SAMPLES_PER_W + t
        pltpu.sync_copy(locs_hbm.at[b], locs_v)

        for k in range(_NP // 16):
            f = k * 16
            l0 = locs_v[f // 128, pl.ds(f % 128, 16)]
            l1 = locs_v[4 + f // 128, pl.ds(f % 128, 16)]
            px = bin_idx(l0)
            py = bin_idx(l1)
            rm = jnp.where(px < 1, px + (_G - 1), px - 1)
            cm = jnp.where(py < 1, py + (_G - 1), py - 1)
            cells = (rm * 1024 + (cm >> 7) * 512 + (cm & 127),
                     rm * 1024 + (py >> 7) * 512 + (py & 127),
                     px * 1024 + (cm >> 7) * 512 + (cm & 127),
                     px * 1024 + (py >> 7) * 512 + (py & 127))
            vals = (one_f, one_f, l0, l1)
            pos = (k * 16 + iota) * 16
            for c4 in range(4):
                for ch in range(4):
                    plsc.store_scatter(addr16, [pos + (c4 * 4 + ch)],
                                       cells[c4] + ch * 128)
                    plsc.store_scatter(vals16, [pos + (c4 * 4 + ch)], vals[ch])

        for q in range(_NSLICE):
            idx = t * _NSLICE + q
            j = idx % 3
            if idx == 0:
                fills[0] = pltpu.async_copy(
                    base_hbm.at[pl.ds(0, _SLICE_ROWS)], bufs[0], fsems[0])
            if idx + 1 < _SAMPLES_PER_W * _NSLICE:
                jn = (idx + 1) % 3
                qn = (idx + 1) % _NSLICE
                if outs[jn] is not None:
                    outs[jn].wait()
                    outs[jn] = None
                fills[jn] = pltpu.async_copy(
                    base_hbm.at[pl.ds(qn * _SLICE_ROWS, _SLICE_ROWS)],
                    bufs[jn], fsems[jn])
            fills[j].wait()
            mapbuf = bufs[j]

            def patch(i, carry, q=q, mapbuf=mapbuf):
                a = addr16[pl.ds(i * 16, 16)]
                v = vals16[pl.ds(i * 16, 16)]
                mask = (a >> 15) == q
                r = (a >> 10) - q * _SLICE_ROWS
                plsc.store_scatter(mapbuf, [r, (a >> 7) & 7, a & 127], v,
                                   mask=mask)
                return carry

            lax.fori_loop(0, _NP, patch, 0)
            outs[j] = pltpu.async_copy(
                mapbuf,
                out_hbm.at[pl.ds(b * _ROWS + q * _SLICE_ROWS, _SLICE_ROWS)],
                osems[j])

    for o in outs:
        if o is not None:
            o.wait()


_INTERPRET = False


def _sc_call(locs, base, btab):
    mesh = plsc.VectorSubcoreMesh(core_axis_name="c", subcore_axis_name="s")
    return pl.kernel(
        _sc_body,
        out_type=jax.ShapeDtypeStruct((_BATCH * _ROWS, 8, 128), jnp.float32),
        mesh=mesh,
        scratch_types=[
            pltpu.VMEM((8, 128), jnp.float32),
            pltpu.VMEM((8, 128), jnp.float32),
            pltpu.VMEM((_NP * 16,), jnp.int32),
            pltpu.VMEM((_NP * 16,), jnp.float32),
            pltpu.VMEM((_SLICE_ROWS, 8, 128), jnp.float32),
            pltpu.VMEM((_SLICE_ROWS, 8, 128), jnp.float32),
            pltpu.VMEM((_SLICE_ROWS, 8, 128), jnp.float32),
            pltpu.SemaphoreType.DMA,
            pltpu.SemaphoreType.DMA,
            pltpu.SemaphoreType.DMA,
            pltpu.SemaphoreType.DMA,
            pltpu.SemaphoreType.DMA,
            pltpu.SemaphoreType.DMA,
        ],
        compiler_params=pltpu.CompilerParams(needs_layout_passes=False),
        interpret=_INTERPRET,
    )(locs, base, btab)


def kernel(batch):
    l0 = jnp.pad(batch[:, :, 0], ((0, 0), (0, 56)), mode="edge")
    l1 = jnp.pad(batch[:, :, 1], ((0, 0), (0, 56)), mode="edge")
    z = jnp.zeros_like(l0)
    locs = jnp.concatenate([l0, z, l1, z], axis=1).reshape(_BATCH, 8, 128)
    base_phys = np.transpose(
        _BASE4.reshape(_G, 2, 128, 4), (0, 1, 3, 2)).reshape(_G, 8, 128)
    out = _sc_call(locs, jnp.asarray(base_phys), jnp.asarray(_BTAB))
    out = out.reshape(_BATCH, _G, 2, 4, 128).transpose(0, 1, 2, 4, 3)
    return out.reshape(_BATCH, _G, _G, 4)


# speedup vs baseline: 1.0955x; 1.0105x over previous
import numpy as np
import jax
import jax.numpy as jnp
from jax import lax
from jax.experimental import pallas as pl
from jax.experimental.pallas import tpu as pltpu
from jax.experimental.pallas import tpu_sc as plsc

_MIN_LOC = np.array([0.0, 0.0], dtype=np.float32)
_MAX_LOC = np.array([1.0, 1.0], dtype=np.float32)
_BINS = np.array([1023.0, 1023.0], dtype=np.float32)
_STRIDE = np.array([4.0, 4.0], dtype=np.float32)
_WINDOW = np.array([1.0, 1.0], dtype=np.float32)
_BATCH, _NLOC = 64, 200


def _build_base():
    window_side = (_WINDOW / 2.0).astype(np.int32).astype(np.float32)
    loc_delta = (_MAX_LOC - _MIN_LOC) / _BINS
    bins_window = _BINS - 2.0 * window_side
    min_window = _MIN_LOC + loc_delta * window_side
    max_window = _MIN_LOC + loc_delta * bins_window
    bins_stride = ((bins_window + 1.0) / _STRIDE).astype(np.int32)
    delta2 = (max_window - min_window) / bins_stride.astype(np.float32)
    xs = np.arange(min_window[0], max_window[0], delta2[0], dtype=np.float32)
    ys = np.arange(min_window[1], max_window[1], delta2[1], dtype=np.float32)
    X, Y = np.meshgrid(xs, ys)
    loc_base = np.stack([X, Y], axis=-1).astype(np.float32)
    corr_base = np.full(loc_base.shape, 0.634, dtype=np.float32)
    base4 = np.concatenate([corr_base, loc_base], axis=-1)
    return base4, loc_delta


_BASE4, _LOC_DELTA = _build_base()
_G = _BASE4.shape[0]
_DL0 = np.float32(_LOC_DELTA[0])
_DL1 = np.float32(_LOC_DELTA[1])
_K = np.float32(255.75)


def _ref_idx(x, dl):
    return ((x / dl) / np.float32(4.0)).astype(np.int32)


def _build_boundaries(dl):
    B = np.zeros(257, dtype=np.float32)
    for k in range(1, 257):
        x = np.float32(np.float64(k) * 4.0 * np.float64(dl))
        while x > 0:
            nx = np.nextafter(x, np.float32(0.0), dtype=np.float32)
            if _ref_idx(nx, dl) >= k:
                x = nx
            else:
                break
        while _ref_idx(x, dl) < k:
            x = np.nextafter(x, np.float32(2.0), dtype=np.float32)
        B[k] = x
    B[0] = 0.0
    return B


def _build_btab():
    assert _DL0 == _DL1 and float(_STRIDE[0]) == float(_STRIDE[1]) == 4.0
    B = _build_boundaries(_DL0)
    tab = np.full((8, 128), 2.0, dtype=np.float32)
    flat = tab.reshape(-1)
    flat[0:257] = B
    flat[512:768] = B[1:]
    return tab


_BTAB = _build_btab()

_NP = 224
_NSLICE = 8
_SLICE_ROWS = _G // _NSLICE
_ROWS = _G
_SAMPLES_PER_W = 2


def _sc_body(locs_hbm, base_hbm, btab_hbm, out_hbm, locs_v, btab_v, addr16,
             vals16, buf0, buf1, buf2, fs0, fs1, fs2, os0, os1, os2):
    bufs = (buf0, buf1, buf2)
    fsems = (fs0, fs1, fs2)
    osems = (os0, os1, os2)
    wid = lax.axis_index("s") * 2 + lax.axis_index("c")
    iota = lax.iota(jnp.int32, 16)
    one_f = jnp.ones((16,), jnp.float32)
    pltpu.sync_copy(btab_hbm, btab_v)

    fills = [None, None, None]
    outs = [None, None, None]

    def bin_idx(x):
        q = (x * _K).astype(jnp.int32)
        lo = plsc.load_gather(btab_v, [q >> 7, q & 127])
        qh = q + 512
        hi = plsc.load_gather(btab_v, [qh >> 7, qh & 127])
        return q + (x >= hi).astype(jnp.int32) - (x < lo).astype(jnp.int32)

    for t in range(_SAMPLES_PER_W):
        b = wid * _SAMPLES_PER_W + t
        pltpu.sync_copy(locs_hbm.at[b], locs_v)

        for k in range(_NP // 16):
            f = k * 16
            l0 = locs_v[f // 128, pl.ds(f % 128, 16)]
            l1 = locs_v[4 + f // 128, pl.ds(f % 128, 16)]
            px = bin_idx(l0)
            py = bin_idx(l1)
            rm = jnp.where(px < 1, px + (_G - 1), px - 1)
            cm = jnp.where(py < 1, py + (_G - 1), py - 1)
            cells = (rm * 1024 + (cm >> 7) * 512 + (cm & 127),
                     rm * 1024 + (py >> 7) * 512 + (py & 127),
                     px * 1024 + (cm >> 7) * 512 + (cm & 127),
                     px * 1024 + (py >> 7) * 512 + (py & 127))
            vals = (one_f, one_f, l0, l1)
            pos = (k * 16 + iota) * 16
            for c4 in range(4):
                for ch in range(4):
                    plsc.store_scatter(addr16, [pos + (c4 * 4 + ch)],
                                       cells[c4] + ch * 128)
                    plsc.store_scatter(vals16, [pos + (c4 * 4 + ch)], vals[ch])

        for q in range(_NSLICE):
            idx = t * _NSLICE + q
            j = idx % 3
            if idx == 0:
                fills[0] = pltpu.async_copy(
                    base_hbm.at[pl.ds(0, _SLICE_ROWS)], bufs[0], fsems[0])
            if idx + 1 < _SAMPLES_PER_W * _NSLICE:
                jn = (idx + 1) % 3
                qn = (idx + 1) % _NSLICE
                if outs[jn] is not None:
                    outs[jn].wait()
                    outs[jn] = None
                fills[jn] = pltpu.async_copy(
                    base_hbm.at[pl.ds(qn * _SLICE_ROWS, _SLICE_ROWS)],
                    bufs[jn], fsems[jn])
            fills[j].wait()
            mapbuf = bufs[j]

            def patch(i, carry, q=q, mapbuf=mapbuf):
                a = addr16[pl.ds(i * 16, 16)]
                v = vals16[pl.ds(i * 16, 16)]
                mask = (a >> 15) == q
                r = (a >> 10) - q * _SLICE_ROWS
                plsc.store_scatter(mapbuf, [r, (a >> 7) & 7, a & 127], v,
                                   mask=mask)
                return carry

            lax.fori_loop(0, _NP // 4, lambda i, c: patch(4 * i + 3, patch(4 * i + 2, patch(4 * i + 1, patch(4 * i, c)))), 0)
            outs[j] = pltpu.async_copy(
                mapbuf,
                out_hbm.at[pl.ds(b * _ROWS + q * _SLICE_ROWS, _SLICE_ROWS)],
                osems[j])

    for o in outs:
        if o is not None:
            o.wait()


_INTERPRET = False


def _sc_call(locs, base, btab):
    mesh = plsc.VectorSubcoreMesh(core_axis_name="c", subcore_axis_name="s")
    return pl.kernel(
        _sc_body,
        out_type=jax.ShapeDtypeStruct((_BATCH * _ROWS, 8, 128), jnp.float32),
        mesh=mesh,
        scratch_types=[
            pltpu.VMEM((8, 128), jnp.float32),
            pltpu.VMEM((8, 128), jnp.float32),
            pltpu.VMEM((_NP * 16,), jnp.int32),
            pltpu.VMEM((_NP * 16,), jnp.float32),
            pltpu.VMEM((_SLICE_ROWS, 8, 128), jnp.float32),
            pltpu.VMEM((_SLICE_ROWS, 8, 128), jnp.float32),
            pltpu.VMEM((_SLICE_ROWS, 8, 128), jnp.float32),
            pltpu.SemaphoreType.DMA,
            pltpu.SemaphoreType.DMA,
            pltpu.SemaphoreType.DMA,
            pltpu.SemaphoreType.DMA,
            pltpu.SemaphoreType.DMA,
            pltpu.SemaphoreType.DMA,
        ],
        compiler_params=pltpu.CompilerParams(needs_layout_passes=False),
        interpret=_INTERPRET,
    )(locs, base, btab)


def kernel(batch):
    l0 = jnp.pad(batch[:, :, 0], ((0, 0), (0, 56)), mode="edge")
    l1 = jnp.pad(batch[:, :, 1], ((0, 0), (0, 56)), mode="edge")
    z = jnp.zeros_like(l0)
    locs = jnp.concatenate([l0, z, l1, z], axis=1).reshape(_BATCH, 8, 128)
    base_phys = np.transpose(
        _BASE4.reshape(_G, 2, 128, 4), (0, 1, 3, 2)).reshape(_G, 8, 128)
    out = _sc_call(locs, jnp.asarray(base_phys), jnp.asarray(_BTAB))
    out = out.reshape(_BATCH, _G, 2, 4, 128).transpose(0, 1, 2, 4, 3)
    return out.reshape(_BATCH, _G, _G, 4)
